# baseline (speedup 1.0000x reference)
import functools

import jax
import jax.numpy as jnp
from jax import lax
from jax.experimental import pallas as pl
from jax.experimental.pallas import tpu as pltpu
from jax.experimental.pallas import tpu_sc as plsc

_COMMITMENT_COST = 0.25
_NT = 256
_NW = 32


def _prep_body(cb_ref, cbm2_ref, c2_ref, cbt_ref):
    cb = cb_ref[...]
    cbm2_ref[...] = (-2.0 * cb).astype(jnp.bfloat16)
    c2_ref[...] = jnp.sum(cb * cb, axis=1, keepdims=True)
    cbt_ref[...] = jnp.swapaxes(cb, 0, 1)


def _vq_body(z_ref, cbm2_ref, c2_ref, idx_ref):
    k = cbm2_ref.shape[0]
    nt = z_ref.shape[2]
    zt = z_ref[0]
    s_neg = jax.lax.dot_general(
        cbm2_ref[...], zt.astype(jnp.bfloat16),
        (((1,), (0,)), ((), ())),
        preferred_element_type=jnp.float32)
    z2 = jnp.sum(zt * zt, axis=0, keepdims=True)
    dist = (z2 + s_neg) + c2_ref[...]
    mins = jnp.min(dist, axis=0, keepdims=True)
    kiota = jax.lax.broadcasted_iota(jnp.int32, (k, nt), 0)
    idx_ref[0] = jnp.min(jnp.where(dist == mins, kiota, k),
                         axis=0, keepdims=True)


def _sc_body(cbt_ref, idx_ref, z_ref, zq_ref, lossp_ref,
             cb_v, idx_v, z_v, out_v, ls_v, *, kk, d, hw, nd):
    nb = z_ref.shape[0] // (d * hw)
    ngrp = hw // 16
    wid = lax.axis_index("s") * 2 + lax.axis_index("c")
    dbase = wid * nd
    pltpu.sync_copy(cbt_ref.at[pl.ds(dbase * kk, nd * kk)], cb_v)
    pltpu.sync_copy(idx_ref, idx_v)
    accs = [jnp.zeros((16,), jnp.float32) for _ in range(nd)]
    for b in range(nb):
        pltpu.sync_copy(
            z_ref.at[pl.ds((b * d + dbase) * hw, nd * hw)], z_v)

        def body(g, accs):
            idxv = idx_v[pl.ds(b * hw + g * 16, 16)]
            out = list(accs)
            for r in range(nd):
                v = plsc.load_gather(cb_v, [idxv + r * kk])
                out_v[pl.ds(r * hw + g * 16, 16)] = v
                dd = v - z_v[pl.ds(r * hw + g * 16, 16)]
                out[r] = out[r] + dd * dd
            return tuple(out)

        accs = list(lax.fori_loop(0, ngrp, body, tuple(accs)))
        pltpu.sync_copy(out_v,
                        zq_ref.at[pl.ds((b * d + dbase) * hw, nd * hw)])
    total = accs[0]
    for r in range(1, nd):
        total = total + accs[r]
    ls_v[...] = total
    pltpu.sync_copy(ls_v, lossp_ref.at[wid])


def kernel(z, codebook):
    b, d, h, w = z.shape
    kk = codebook.shape[0]
    hw = h * w
    nt = min(_NT, hw)
    nblk = hw // nt
    grid = b * nblk
    z3 = z.reshape(b, d, hw)

    cbm2, c2, cbt = pl.pallas_call(
        _prep_body,
        out_shape=[
            jax.ShapeDtypeStruct((kk, d), jnp.bfloat16),
            jax.ShapeDtypeStruct((kk, 1), jnp.float32),
            jax.ShapeDtypeStruct((d, kk), jnp.float32),
        ],
    )(codebook)

    idx = pl.pallas_call(
        _vq_body,
        grid=(grid,),
        in_specs=[
            pl.BlockSpec((1, d, nt), lambda i: (i // nblk, 0, i % nblk)),
            pl.BlockSpec((kk, d), lambda i: (0, 0)),
            pl.BlockSpec((kk, 1), lambda i: (0, 0)),
        ],
        out_specs=pl.BlockSpec((1, 1, nt), lambda i: (i, 0, 0)),
        out_shape=jax.ShapeDtypeStruct((grid, 1, nt), jnp.int32),
    )(z3, cbm2, c2)

    nd = d // _NW
    sc = pl.kernel(
        functools.partial(_sc_body, kk=kk, d=d, hw=hw, nd=nd),
        out_type=[
            jax.ShapeDtypeStruct((b * d * hw,), jnp.float32),
            jax.ShapeDtypeStruct((_NW, 16), jnp.float32),
        ],
        mesh=plsc.VectorSubcoreMesh(core_axis_name="c", subcore_axis_name="s"),
        compiler_params=pltpu.CompilerParams(needs_layout_passes=False),
        scratch_types=[
            pltpu.VMEM((nd * kk,), jnp.float32),
            pltpu.VMEM((b * hw,), jnp.int32),
            pltpu.VMEM((nd * hw,), jnp.float32),
            pltpu.VMEM((nd * hw,), jnp.float32),
            pltpu.VMEM((16,), jnp.float32),
        ],
    )
    zq1, lossp = sc(cbt.reshape(-1), idx.reshape(-1), z3.reshape(-1))

    zq_out = zq1.reshape(b, d, h, w)
    idx_out = idx.reshape(b, h, w)
    mse = jnp.sum(lossp) / (b * d * hw)
    vq_loss = mse + _COMMITMENT_COST * mse
    return zq_out, idx_out, vq_loss


# speedup vs baseline: 1.3190x; 1.3190x over previous
import functools

import jax
import jax.numpy as jnp
from jax import lax
from jax.experimental import pallas as pl
from jax.experimental.pallas import tpu as pltpu
from jax.experimental.pallas import tpu_sc as plsc

_COMMITMENT_COST = 0.25
_NT = 256
_NW = 32


def _prep_body(cb_ref, cbm2_ref, c2_ref, cbt_ref):
    cb = cb_ref[...]
    cbm2_ref[...] = (-2.0 * cb).astype(jnp.bfloat16)
    c2_ref[...] = jnp.sum(cb * cb, axis=1, keepdims=True)
    cbt_ref[...] = jnp.swapaxes(cb, 0, 1)


def _vq_body(z_ref, cbm2_ref, c2_ref, idx_ref, ls_ref):
    k = cbm2_ref.shape[0]
    nt = z_ref.shape[2]
    zt = z_ref[0]
    s_neg = jax.lax.dot_general(
        cbm2_ref[...], zt.astype(jnp.bfloat16),
        (((1,), (0,)), ((), ())),
        preferred_element_type=jnp.float32)
    z2 = jnp.sum(zt * zt, axis=0, keepdims=True)
    dist = (z2 + s_neg) + c2_ref[...]
    mins = jnp.min(dist, axis=0, keepdims=True)
    kiota = jax.lax.broadcasted_iota(jnp.int32, (k, nt), 0)
    idx_ref[0] = jnp.min(jnp.where(dist == mins, kiota, k),
                         axis=0, keepdims=True)

    @pl.when(pl.program_id(0) == 0)
    def _init():
        ls_ref[...] = jnp.zeros_like(ls_ref)

    ls_ref[...] += mins


def _sc_body(cbt_ref, idx_ref, zq_ref, cb_v, idx_v, out_v, *, kk, d, hw, nd):
    nb = zq_ref.shape[0] // d
    ngrp = hw // 16
    wid = lax.axis_index("s") * 2 + lax.axis_index("c")
    dbase = wid * nd
    for r in range(nd):
        pltpu.sync_copy(cbt_ref.at[dbase + r], cb_v.at[pl.ds(r * kk, kk)])
    pltpu.sync_copy(idx_ref, idx_v)
    for b in range(nb):

        def body(g, carry):
            idxv = idx_v[pl.ds(b * hw + g * 16, 16)]
            for r in range(nd):
                v = plsc.load_gather(cb_v, [idxv + r * kk])
                out_v[r, pl.ds(g * 16, 16)] = v
            return carry

        lax.fori_loop(0, ngrp, body, 0)
        pltpu.sync_copy(out_v, zq_ref.at[pl.ds(b * d + dbase, nd)])


def kernel(z, codebook):
    b, d, h, w = z.shape
    kk = codebook.shape[0]
    hw = h * w
    nt = min(_NT, hw)
    nblk = hw // nt
    grid = b * nblk
    z3 = z.reshape(b, d, hw)

    cbm2, c2, cbt = pl.pallas_call(
        _prep_body,
        out_shape=[
            jax.ShapeDtypeStruct((kk, d), jnp.bfloat16),
            jax.ShapeDtypeStruct((kk, 1), jnp.float32),
            jax.ShapeDtypeStruct((d, kk), jnp.float32),
        ],
    )(codebook)

    idx, ls = pl.pallas_call(
        _vq_body,
        grid=(grid,),
        in_specs=[
            pl.BlockSpec((1, d, nt), lambda i: (i // nblk, 0, i % nblk)),
            pl.BlockSpec((kk, d), lambda i: (0, 0)),
            pl.BlockSpec((kk, 1), lambda i: (0, 0)),
        ],
        out_specs=[
            pl.BlockSpec((1, 1, nt), lambda i: (i, 0, 0)),
            pl.BlockSpec((1, nt), lambda i: (0, 0)),
        ],
        out_shape=[
            jax.ShapeDtypeStruct((grid, 1, nt), jnp.int32),
            jax.ShapeDtypeStruct((1, nt), jnp.float32),
        ],
    )(z3, cbm2, c2)

    nd = d // _NW
    sc = pl.kernel(
        functools.partial(_sc_body, kk=kk, d=d, hw=hw, nd=nd),
        out_type=jax.ShapeDtypeStruct((b * d, hw), jnp.float32),
        mesh=plsc.VectorSubcoreMesh(core_axis_name="c", subcore_axis_name="s"),
        compiler_params=pltpu.CompilerParams(needs_layout_passes=False),
        scratch_types=[
            pltpu.VMEM((nd * kk,), jnp.float32),
            pltpu.VMEM((b * hw,), jnp.int32),
            pltpu.VMEM((nd, hw), jnp.float32),
        ],
    )
    zq2 = sc(cbt, idx.reshape(-1))

    zq_out = zq2.reshape(b, d, h, w)
    idx_out = idx.reshape(b, h, w)
    mse = jnp.sum(ls) / (b * d * hw)
    vq_loss = mse + _COMMITMENT_COST * mse
    return zq_out, idx_out, vq_loss


# speedup vs baseline: 1.7364x; 1.3164x over previous
import functools

import jax
import jax.numpy as jnp
from jax import lax
from jax.experimental import pallas as pl
from jax.experimental.pallas import tpu as pltpu
from jax.experimental.pallas import tpu_sc as plsc

_COMMITMENT_COST = 0.25
_NT = 256
_NW = 32


def _prep_body(cb_ref, cbm2_ref, c2_ref, cbt_ref):
    cb = cb_ref[...]
    cbm2_ref[...] = (-2.0 * cb).astype(jnp.bfloat16)
    c2_ref[...] = jnp.sum(cb * cb, axis=1, keepdims=True)
    cbt_ref[...] = jnp.swapaxes(cb, 0, 1)


def _vq_body(z_ref, cbm2_ref, c2_ref, idx_ref, ls_ref):
    k = cbm2_ref.shape[0]
    nt = z_ref.shape[2]
    zt = z_ref[0]
    s_neg = jax.lax.dot_general(
        cbm2_ref[...], zt.astype(jnp.bfloat16),
        (((1,), (0,)), ((), ())),
        preferred_element_type=jnp.float32)
    z2 = jnp.sum(zt * zt, axis=0, keepdims=True)
    dist = (z2 + s_neg) + c2_ref[...]
    mins = jnp.min(dist, axis=0, keepdims=True)
    idx_ref[0] = jnp.argmin(dist, axis=0)[None, :]

    @pl.when(pl.program_id(0) == 0)
    def _init():
        ls_ref[...] = jnp.zeros_like(ls_ref)

    ls_ref[...] += mins


def _sc_body(cbt_ref, idx_ref, zq_ref, cb_v, idx_v, out_v, *, kk, d, hw, nd):
    nb = zq_ref.shape[0]
    ngrp = hw // 16
    wid = lax.axis_index("s") * 2 + lax.axis_index("c")
    dbase = wid * nd
    for r in range(nd):
        pltpu.sync_copy(cbt_ref.at[dbase + r], cb_v.at[pl.ds(r * kk, kk)])
    pltpu.sync_copy(idx_ref, idx_v)
    for b in range(nb):

        def body(g, carry):
            idxv = idx_v[pl.ds(b * hw + g * 16, 16)]
            for r in range(nd):
                v = plsc.load_gather(cb_v, [idxv + r * kk])
                out_v[r, pl.ds(g * 16, 16)] = v
            return carry

        lax.fori_loop(0, ngrp, body, 0)
        pltpu.sync_copy(out_v, zq_ref.at[b, pl.ds(dbase, nd)])


def kernel(z, codebook):
    b, d, h, w = z.shape
    kk = codebook.shape[0]
    hw = h * w
    nt = min(_NT, hw)
    nblk = hw // nt
    grid = b * nblk
    z3 = z.reshape(b, d, hw)

    cbm2, c2, cbt = pl.pallas_call(
        _prep_body,
        out_shape=[
            jax.ShapeDtypeStruct((kk, d), jnp.bfloat16),
            jax.ShapeDtypeStruct((kk, 1), jnp.float32),
            jax.ShapeDtypeStruct((d, kk), jnp.float32),
        ],
    )(codebook)

    idx, ls = pl.pallas_call(
        _vq_body,
        grid=(grid,),
        in_specs=[
            pl.BlockSpec((1, d, nt), lambda i: (i // nblk, 0, i % nblk)),
            pl.BlockSpec((kk, d), lambda i: (0, 0)),
            pl.BlockSpec((kk, 1), lambda i: (0, 0)),
        ],
        out_specs=[
            pl.BlockSpec((1, 1, nt), lambda i: (i, 0, 0)),
            pl.BlockSpec((1, nt), lambda i: (0, 0)),
        ],
        out_shape=[
            jax.ShapeDtypeStruct((grid, 1, nt), jnp.int32),
            jax.ShapeDtypeStruct((1, nt), jnp.float32),
        ],
    )(z3, cbm2, c2)

    nd = d // _NW
    sc = pl.kernel(
        functools.partial(_sc_body, kk=kk, d=d, hw=hw, nd=nd),
        out_type=jax.ShapeDtypeStruct((b, d, hw), jnp.float32),
        mesh=plsc.VectorSubcoreMesh(core_axis_name="c", subcore_axis_name="s"),
        compiler_params=pltpu.CompilerParams(needs_layout_passes=False),
        scratch_types=[
            pltpu.VMEM((nd * kk,), jnp.float32),
            pltpu.VMEM((b * hw,), jnp.int32),
            pltpu.VMEM((nd, hw), jnp.float32),
        ],
    )
    zq2 = sc(cbt, idx.reshape(-1))

    zq_out = zq2.reshape(b, d, h, w)
    idx_out = idx.reshape(b, h, w)
    mse = jnp.sum(ls) / (b * d * hw)
    vq_loss = mse + _COMMITMENT_COST * mse
    return zq_out, idx_out, vq_loss


# speedup vs baseline: 2.0982x; 1.2083x over previous
import functools

import jax
import jax.numpy as jnp
from jax import lax
from jax.experimental import pallas as pl
from jax.experimental.pallas import tpu as pltpu
from jax.experimental.pallas import tpu_sc as plsc

_COMMITMENT_COST = 0.25
_NT = 256
_NW = 32


def _prep_body(cb_ref, cbm2_ref, c2_ref, cbt_ref):
    cb = cb_ref[...]
    cbm2_ref[...] = (-2.0 * cb).astype(jnp.bfloat16)
    c2_ref[...] = jnp.sum(cb * cb, axis=1, keepdims=True)
    cbt_ref[...] = jnp.swapaxes(cb, 0, 1)


def _vq_body(z_ref, cbm2_ref, c2_ref, idx_ref, ls_ref):
    k = cbm2_ref.shape[0]
    nt = z_ref.shape[2]
    zt = z_ref[0]
    s_neg = jax.lax.dot_general(
        cbm2_ref[...], zt.astype(jnp.bfloat16),
        (((1,), (0,)), ((), ())),
        preferred_element_type=jnp.float32)
    z2 = jnp.sum(zt * zt, axis=0, keepdims=True)
    dist = (z2 + s_neg) + c2_ref[...]
    mins = jnp.min(dist, axis=0, keepdims=True)
    idx_ref[0] = jnp.argmin(dist, axis=0)[None, :]

    @pl.when(pl.program_id(0) == 0)
    def _init():
        ls_ref[...] = jnp.zeros_like(ls_ref)

    ls_ref[...] += mins


def _sc_body(cbt_ref, idx_ref, zq_ref, cb_v, idx_v, out_v, sem0, sem1,
             *, kk, d, hw, nd):
    nb = zq_ref.shape[0]
    ngrp = hw // 16
    wid = lax.axis_index("s") * 2 + lax.axis_index("c")
    dbase = wid * nd
    for r in range(nd):
        pltpu.sync_copy(cbt_ref.at[dbase + r], cb_v.at[pl.ds(r * kk, kk)])
    pltpu.sync_copy(idx_ref, idx_v)
    sems = (sem0, sem1)
    handles = [None, None]
    for b in range(nb):
        buf = b % 2
        if handles[buf] is not None:
            handles[buf].wait()

        @plsc.parallel_loop(0, ngrp, unroll=8)
        def body(g):
            idxv = idx_v[pl.ds(b * hw + g * 16, 16)]
            for r in range(nd):
                v = plsc.load_gather(cb_v, [idxv + r * kk])
                out_v[buf, r, pl.ds(g * 16, 16)] = v

        handles[buf] = pltpu.async_copy(
            out_v.at[buf], zq_ref.at[b, pl.ds(dbase, nd)], sems[buf])
    handles[0].wait()
    handles[1].wait()


def kernel(z, codebook):
    b, d, h, w = z.shape
    kk = codebook.shape[0]
    hw = h * w
    nt = min(_NT, hw)
    nblk = hw // nt
    grid = b * nblk
    z3 = z.reshape(b, d, hw)

    cbm2, c2, cbt = pl.pallas_call(
        _prep_body,
        out_shape=[
            jax.ShapeDtypeStruct((kk, d), jnp.bfloat16),
            jax.ShapeDtypeStruct((kk, 1), jnp.float32),
            jax.ShapeDtypeStruct((d, kk), jnp.float32),
        ],
    )(codebook)

    idx, ls = pl.pallas_call(
        _vq_body,
        grid=(grid,),
        in_specs=[
            pl.BlockSpec((1, d, nt), lambda i: (i // nblk, 0, i % nblk)),
            pl.BlockSpec((kk, d), lambda i: (0, 0)),
            pl.BlockSpec((kk, 1), lambda i: (0, 0)),
        ],
        out_specs=[
            pl.BlockSpec((1, 1, nt), lambda i: (i, 0, 0)),
            pl.BlockSpec((1, nt), lambda i: (0, 0)),
        ],
        out_shape=[
            jax.ShapeDtypeStruct((grid, 1, nt), jnp.int32),
            jax.ShapeDtypeStruct((1, nt), jnp.float32),
        ],
    )(z3, cbm2, c2)

    nd = d // _NW
    sc = pl.kernel(
        functools.partial(_sc_body, kk=kk, d=d, hw=hw, nd=nd),
        out_type=jax.ShapeDtypeStruct((b, d, hw), jnp.float32),
        mesh=plsc.VectorSubcoreMesh(core_axis_name="c", subcore_axis_name="s"),
        compiler_params=pltpu.CompilerParams(needs_layout_passes=False),
        scratch_types=[
            pltpu.VMEM((nd * kk,), jnp.float32),
            pltpu.VMEM((b * hw,), jnp.int32),
            pltpu.VMEM((2, nd, hw), jnp.float32),
            pltpu.SemaphoreType.DMA,
            pltpu.SemaphoreType.DMA,
        ],
    )
    zq2 = sc(cbt, idx.reshape(-1))

    zq_out = zq2.reshape(b, d, h, w)
    idx_out = idx.reshape(b, h, w)
    mse = jnp.sum(ls) / (b * d * hw)
    vq_loss = mse + _COMMITMENT_COST * mse
    return zq_out, idx_out, vq_loss
